# baseline (speedup 1.0000x reference)
import functools

import jax
import jax.numpy as jnp
from jax import lax
from jax.experimental import pallas as pl
from jax.experimental.pallas import tpu as pltpu
from jax.experimental.pallas import tpu_sc as plsc
from jax._src.pallas import mpmd as _pl_mpmd

_LANES = 16
_CHUNK = 128
_CP_BLOCK = 4000


def _tc_copy_body(x_ref, o_ref):
  o_ref[...] = x_ref[...]


def _prep_body(idx_hbm, frows_hbm, fpos_hbm, cnts_hbm,
               idx_v, winner, frows, fpos, cbuf,
               *, num_cores, rows_per_tile, batch, npad):
  wid = lax.axis_index("s") * num_cores + lax.axis_index("c")
  base = wid * rows_per_tile
  ngroups = batch // _LANES
  nwin = (rows_per_tile + _LANES - 1) // _LANES

  pltpu.sync_copy(idx_hbm, idx_v)

  lanes = lax.iota(jnp.int32, _LANES)
  zeros = jnp.zeros((_LANES,), jnp.int32)
  neg1 = zeros - 1

  def init_step(k, _):
    winner[pl.ds(k * _LANES, _LANES)] = neg1
    return 0

  lax.fori_loop(0, nwin, init_step, 0)

  def scan_group(g, _):
    iv = idx_v[pl.ds(g * _LANES, _LANES)]
    lr = iv - base
    m = (lr >= 0) & (lr < rows_per_tile)
    _, last_m = plsc.scan_count(lr, mask=m)
    plsc.store_scatter(winner, [lr], g * _LANES + lanes, mask=last_m)
    return 0

  lax.fori_loop(0, ngroups, scan_group, 0)

  def compact_step(k, cnt2):
    w = winner[pl.ds(k * _LANES, _LANES)]
    keep = w >= 0
    offs = cnt2 + plsc.cumsum(jnp.where(keep, 1, 0)) - 1
    plsc.store_scatter(frows, [offs], k * _LANES + lanes, mask=keep)
    plsc.store_scatter(fpos, [offs], w, mask=keep)
    return cnt2 + plsc.all_reduce_population_count(keep)

  cnt2 = lax.fori_loop(0, nwin, compact_step, zeros)

  cbuf[pl.ds(0, _LANES)] = cnt2
  pltpu.sync_copy(frows, frows_hbm.at[wid])
  pltpu.sync_copy(fpos, fpos_hbm.at[wid])
  pltpu.sync_copy(cbuf, cnts_hbm.at[wid])


def _apply_body(vals_hbm, frows_hbm, fpos_hbm, cnts_hbm, out_in, out_hbm,
                frows, fpos, cbuf,
                sr0, sr1, sr2, sr3, sp0, sp1, sp2, sp3,
                rb0, rb1, rb2, rb3, gs0, gs1, gs2, gs3,
                ss0, ss1, ss2, ss3,
                *, num_cores, rows_per_tile):
  del out_in
  wid = lax.axis_index("s") * num_cores + lax.axis_index("c")
  base = wid * rows_per_tile

  pltpu.sync_copy(frows_hbm.at[wid], frows)
  pltpu.sync_copy(fpos_hbm.at[wid], fpos)
  pltpu.sync_copy(cnts_hbm.at[wid], cbuf)

  lanes = lax.iota(jnp.int32, _LANES)
  cnt2 = cbuf[pl.ds(0, _LANES)]
  cnt2_s = jnp.max(cnt2)

  srows = (sr0, sr1, sr2, sr3)
  spos = (sp0, sp1, sp2, sp3)
  rbufs = (rb0, rb1, rb2, rb3)
  gsems = (gs0, gs1, gs2, gs3)
  ssems = (ss0, ss1, ss2, ss3)

  def g_desc(u):
    return pltpu.make_async_copy(vals_hbm.at[spos[u]], rbufs[u], gsems[u])

  def s_desc(u):
    return pltpu.make_async_copy(rbufs[u], out_hbm.at[srows[u]], ssems[u])

  @pl.when(cnt2_s > 0)
  def _():
    last = jnp.maximum(cnt2 - 1, 0)
    last_r = plsc.load_gather(frows, [last])
    last_p = plsc.load_gather(fpos, [last])
    nchunks = (cnt2_s + _CHUNK - 1) // _CHUNK

    def fill(u, j):
      for k in range(_CHUNK // _LANES):
        st = j * _CHUNK + k * _LANES
        gid = st + lanes
        valid = gid < cnt2
        r = jnp.where(valid, frows[pl.ds(st, _LANES)], last_r)
        p = jnp.where(valid, fpos[pl.ds(st, _LANES)], last_p)
        srows[u][pl.ds(k * _LANES, _LANES)] = r + base
        spos[u][pl.ds(k * _LANES, _LANES)] = p

    for u in range(4):
      @pl.when(u < nchunks)
      def _(u=u):
        fill(u, u)
        g_desc(u).start()

    def chunk_quad(qq, _):
      for u in range(4):
        j = 4 * qq + u

        @pl.when(j < nchunks)
        def _(u=u, j=j):
          g_desc(u).wait()
          s_desc(u).start()

          @pl.when(j + 4 < nchunks)
          def _(u=u, j=j):
            s_desc(u).wait()
            fill(u, j + 4)
            g_desc(u).start()
      return 0

    lax.fori_loop(0, (nchunks + 3) // 4, chunk_quad, 0)

    for u in range(4):
      @pl.when(u < nchunks)
      def _(u=u):
        s_desc(u).wait()


def kernel(x, indices, values):
  m, d = x.shape
  b = indices.shape[0]
  idx = indices.astype(jnp.int32)
  info = plsc.get_sparse_core_info()
  nw = info.num_cores * info.num_subcores
  rows_per_tile = m // nw
  npad = ((rows_per_tile + _CHUNK - 1) // _CHUNK) * _CHUNK
  assert m % nw == 0 and b % _LANES == 0 and m % _CP_BLOCK == 0

  tc_copy = pl.pallas_call(
      _tc_copy_body,
      grid=(m // _CP_BLOCK,),
      in_specs=[pl.BlockSpec((_CP_BLOCK, d), lambda i: (i, 0))],
      out_specs=pl.BlockSpec((_CP_BLOCK, d), lambda i: (i, 0)),
      out_shape=jax.ShapeDtypeStruct((m, d), jnp.float32),
  )

  mesh = plsc.VectorSubcoreMesh(core_axis_name="c", subcore_axis_name="s")
  sc_params = pltpu.CompilerParams(use_tc_tiling_on_sc=False,
                                   needs_layout_passes=False)

  sc_prep = pl.kernel(
      functools.partial(_prep_body, num_cores=info.num_cores,
                        rows_per_tile=rows_per_tile, batch=b, npad=npad),
      out_type=(jax.ShapeDtypeStruct((nw, npad), jnp.int32),
                jax.ShapeDtypeStruct((nw, npad), jnp.int32),
                jax.ShapeDtypeStruct((nw, _LANES), jnp.int32)),
      mesh=mesh,
      compiler_params=sc_params,
      scratch_types=[
          pltpu.VMEM((b,), jnp.int32),
          pltpu.VMEM((npad,), jnp.int32),
          pltpu.VMEM((npad,), jnp.int32),
          pltpu.VMEM((npad,), jnp.int32),
          pltpu.VMEM((_LANES,), jnp.int32),
      ],
  )

  sc_apply = _pl_mpmd._mpmd_map(
      [(mesh, functools.partial(_apply_body, num_cores=info.num_cores,
                                rows_per_tile=rows_per_tile))],
      out_types=jax.ShapeDtypeStruct((m, d), jnp.float32),
      input_output_aliases={4: 0},
      compiler_params=sc_params,
      scratch_types=[
          pltpu.VMEM((npad,), jnp.int32),
          pltpu.VMEM((npad,), jnp.int32),
          pltpu.VMEM((_LANES,), jnp.int32),
      ] + [pltpu.VMEM((_CHUNK,), jnp.int32)] * 8
        + [pltpu.VMEM((_CHUNK, d), jnp.float32)] * 4
        + [pltpu.SemaphoreType.DMA] * 8,
  )

  out0 = tc_copy(x)
  fr, fp, cn = sc_prep(idx)
  return sc_apply(values, fr, fp, cn, out0)
